# baseline (speedup 1.0000x reference)
import functools

import jax
import jax.numpy as jnp
from jax import lax
from jax.experimental import pallas as pl
from jax.experimental.pallas import tpu as pltpu
from jax.experimental.pallas import tpu_sc as plsc

_NUM_ENTRIES = 8192
_ENTRY_DIM = 32
_N_TOKENS = 8192
_N_BLK = 512


def _argmax_body(zt_ref, w_ref, idx_ref):
    w = w_ref[...]
    zb = zt_ref[...].T
    dot2 = lax.dot_general(w, zb + zb, (((1,), (0,)), ((), ())))
    z2 = jnp.sum(zb * zb, axis=0, keepdims=True)
    c2 = jnp.sum(w * w, axis=1, keepdims=True)
    t3 = (z2 + c2) - dot2
    idx_ref[0, :] = jnp.argmin(t3, axis=0).astype(jnp.int32)


def _nearest_indices(zt, w):
    grid = (_N_TOKENS // _N_BLK,)
    return pl.pallas_call(
        _argmax_body,
        grid=grid,
        in_specs=[
            pl.BlockSpec((_N_BLK, _ENTRY_DIM), lambda i: (i, 0)),
            pl.BlockSpec((_NUM_ENTRIES, _ENTRY_DIM), lambda i: (0, 0)),
        ],
        out_specs=pl.BlockSpec((1, _N_BLK), lambda i: (0, i)),
        out_shape=jax.ShapeDtypeStruct((1, _N_TOKENS), jnp.int32),
    )(zt, w)


def _make_gather():
    info = plsc.get_sparse_core_info()
    nw = info.num_cores * info.num_subcores
    b_per_w = _N_TOKENS // nw
    mesh = plsc.VectorSubcoreMesh(core_axis_name="c", subcore_axis_name="s")

    @functools.partial(
        pl.kernel,
        mesh=mesh,
        compiler_params=pltpu.CompilerParams(use_tc_tiling_on_sc=False),
        out_type=jax.ShapeDtypeStruct((_N_TOKENS, _ENTRY_DIM), jnp.float32),
        scratch_types=[
            pltpu.VMEM((b_per_w,), jnp.int32),
            pltpu.VMEM((b_per_w, _ENTRY_DIM), jnp.float32),
            pltpu.SemaphoreType.DMA,
        ],
    )
    def gather(table_hbm, idx_hbm, out_hbm, idx_v, rows_v, sem):
        wid = lax.axis_index("s") * info.num_cores + lax.axis_index("c")
        base = wid * b_per_w
        pltpu.sync_copy(idx_hbm.at[pl.ds(base, b_per_w)], idx_v)
        pltpu.async_copy(table_hbm.at[idx_v], rows_v, sem).wait()
        pltpu.sync_copy(rows_v, out_hbm.at[pl.ds(base, b_per_w)])

    return gather


def kernel(z, W):
    input_shape = z.shape
    zt = z.reshape(-1, _ENTRY_DIM)
    idx_flat = _nearest_indices(zt, W).reshape(_N_TOKENS)
    z_q = _make_gather()(W, idx_flat)
    z_q_st = z_q.reshape(input_shape)
    indices = idx_flat.reshape(*input_shape[:-1], 1)
    commitment_loss = jnp.zeros((1,), dtype=z.dtype)
    return (z_q_st, indices, commitment_loss)


# speedup vs baseline: 1.0871x; 1.0871x over previous
import functools

import jax
import jax.numpy as jnp
from jax import lax
from jax.experimental import pallas as pl
from jax.experimental.pallas import tpu as pltpu
from jax.experimental.pallas import tpu_sc as plsc

_NUM_ENTRIES = 8192
_ENTRY_DIM = 32
_N_TOKENS = 8192
_N_BLK = 512


def _argmax_body(zt_ref, w_ref, idx_ref):
    w = w_ref[...]
    zb = zt_ref[...]
    dot2 = lax.dot_general(w, zb + zb, (((1,), (0,)), ((), ())))
    z2 = jnp.sum(zb * zb, axis=0, keepdims=True)
    c2 = jnp.sum(w * w, axis=1, keepdims=True)
    t3 = (z2 + c2) - dot2
    idx_ref[0, :] = jnp.argmin(t3, axis=0).astype(jnp.int32)


def _nearest_indices(zt, w):
    grid = (_N_TOKENS // _N_BLK,)
    return pl.pallas_call(
        _argmax_body,
        grid=grid,
        in_specs=[
            pl.BlockSpec((_ENTRY_DIM, _N_BLK), lambda i: (0, i)),
            pl.BlockSpec((_NUM_ENTRIES, _ENTRY_DIM), lambda i: (0, 0)),
        ],
        out_specs=pl.BlockSpec((1, _N_BLK), lambda i: (0, i)),
        out_shape=jax.ShapeDtypeStruct((1, _N_TOKENS), jnp.int32),
    )(zt, w)


def _make_gather():
    info = plsc.get_sparse_core_info()
    nw = info.num_cores * info.num_subcores
    b_per_w = _N_TOKENS // nw
    mesh = plsc.VectorSubcoreMesh(core_axis_name="c", subcore_axis_name="s")

    @functools.partial(
        pl.kernel,
        mesh=mesh,
        compiler_params=pltpu.CompilerParams(use_tc_tiling_on_sc=False),
        out_type=jax.ShapeDtypeStruct((_N_TOKENS, _ENTRY_DIM), jnp.float32),
        scratch_types=[
            pltpu.VMEM((b_per_w,), jnp.int32),
            pltpu.VMEM((b_per_w, _ENTRY_DIM), jnp.float32),
            pltpu.SemaphoreType.DMA,
        ],
    )
    def gather(table_hbm, idx_hbm, out_hbm, idx_v, rows_v, sem):
        wid = lax.axis_index("s") * info.num_cores + lax.axis_index("c")
        base = wid * b_per_w
        pltpu.sync_copy(idx_hbm.at[pl.ds(base, b_per_w)], idx_v)
        pltpu.async_copy(table_hbm.at[idx_v], rows_v, sem).wait()
        pltpu.sync_copy(rows_v, out_hbm.at[pl.ds(base, b_per_w)])

    return gather


def kernel(z, W):
    input_shape = z.shape
    zt = z.reshape(-1, _ENTRY_DIM).T
    idx_flat = _nearest_indices(zt, W).reshape(_N_TOKENS)
    z_q = _make_gather()(W, idx_flat)
    z_q_st = z_q.reshape(input_shape)
    indices = idx_flat.reshape(*input_shape[:-1], 1)
    commitment_loss = jnp.zeros((1,), dtype=z.dtype)
    return (z_q_st, indices, commitment_loss)
